# Initial kernel scaffold; baseline (speedup 1.0000x reference)
import jax
import jax.numpy as jnp
from jax.experimental import pallas as pl


def kernel(x, edge_index, edge_attr, We0, be0, W10, b10, g0, bt0, W20, b20, We1, be1, W11, b11, g1, bt1, W21, b21, Wc1, bc1, Wc2, bc2):
    raise NotImplementedError("write your pallas kernel here")


# speedup vs baseline: 2.7195x; 2.7195x over previous
import functools

import jax
import jax.numpy as jnp
from jax import lax
from jax.experimental import pallas as pl
from jax.experimental.pallas import tpu as pltpu
from jax.experimental.pallas import tpu_sc as plsc

N = 10000
E = 320000
D = 128
DE = 16
BN_EPS = 1e-5

NC = 2
NS = 16
EPT = E // (NC * NS)
EB = 80
NBLK = EPT // EB
NP = 10240
RPT = NP // NS


def _sc_agg_body(x_hbm, src_hbm, dst_hbm, e_hbm, zeros_hbm, out_hbm,
                 agg_sh, src_v, dst_v, xbuf, ebuf, sem):
  c = lax.axis_index("c")
  s = lax.axis_index("s")

  pltpu.sync_copy(zeros_hbm.at[:], agg_sh.at[pl.ds(s * RPT, RPT), :])
  plsc.subcore_barrier()

  base0 = (c * NS + s) * EPT

  def body(i, carry):
    base = base0 + i * EB
    pltpu.sync_copy(src_hbm.at[pl.ds(base, EB)], src_v)
    g = pltpu.async_copy(x_hbm.at[src_v], xbuf, sem)
    pltpu.sync_copy(e_hbm.at[pl.ds(base, EB), :], ebuf)
    pltpu.sync_copy(dst_hbm.at[pl.ds(base, EB)], dst_v)
    g.wait()

    def row(r, carry2):
      for k in range(D // 16):
        sl = pl.ds(k * 16, 16)
        v = xbuf[r, sl] + ebuf[r, sl]
        xbuf[r, sl] = jnp.maximum(v, 0.0)
      return carry2

    lax.fori_loop(0, EB, row, 0, unroll=False)
    pltpu.sync_copy(xbuf, agg_sh.at[dst_v], add=True)
    return carry

  lax.fori_loop(0, NBLK, body, 0, unroll=False)
  plsc.subcore_barrier()
  pltpu.sync_copy(agg_sh.at[pl.ds(s * RPT, RPT), :],
                  out_hbm.at[c, pl.ds(s * RPT, RPT), :])


@jax.jit
def _sc_agg(x, src, dst, e):
  zeros = jnp.zeros((RPT, D), jnp.float32)
  fn = pl.kernel(
      _sc_agg_body,
      out_type=jax.ShapeDtypeStruct((NC, NP, D), jnp.float32),
      mesh=plsc.VectorSubcoreMesh(core_axis_name="c", subcore_axis_name="s"),
      scratch_types=[
          pltpu.VMEM_SHARED((NP, D), jnp.float32),
          pltpu.VMEM((EB,), jnp.int32),
          pltpu.VMEM((EB,), jnp.int32),
          pltpu.VMEM((EB, D), jnp.float32),
          pltpu.VMEM((EB, D), jnp.float32),
          pltpu.SemaphoreType.DMA,
      ],
  )
  return fn(x, src, dst, e, zeros)


_EBLK = 2000
_RBLK = 1000


def _emb_kernel(ea_ref, we0_ref, be0_ref, we1_ref, be1_ref, e0_ref, e1_ref):
  ea = ea_ref[...]
  e0_ref[...] = jnp.dot(ea, we0_ref[...],
                        preferred_element_type=jnp.float32) + be0_ref[...]
  e1_ref[...] = jnp.dot(ea, we1_ref[...],
                        preferred_element_type=jnp.float32) + be1_ref[...]


@jax.jit
def _emb(edge_attr, We0, be0, We1, be1):
  grid = (E // _EBLK,)
  return pl.pallas_call(
      _emb_kernel,
      grid=grid,
      in_specs=[
          pl.BlockSpec((_EBLK, DE), lambda i: (i, 0)),
          pl.BlockSpec((DE, D), lambda i: (0, 0)),
          pl.BlockSpec((1, D), lambda i: (0, 0)),
          pl.BlockSpec((DE, D), lambda i: (0, 0)),
          pl.BlockSpec((1, D), lambda i: (0, 0)),
      ],
      out_specs=[
          pl.BlockSpec((_EBLK, D), lambda i: (i, 0)),
          pl.BlockSpec((_EBLK, D), lambda i: (i, 0)),
      ],
      out_shape=[
          jax.ShapeDtypeStruct((E, D), jnp.float32),
          jax.ShapeDtypeStruct((E, D), jnp.float32),
      ],
  )(edge_attr, We0, be0.reshape(1, D), We1, be1.reshape(1, D))


def _p1_kernel(x_ref, a0_ref, a1_ref, w_ref, b_ref, t_ref, st_ref):
  i = pl.program_id(0)
  h = x_ref[...] + a0_ref[0] + a1_ref[0]
  t = jnp.dot(h, w_ref[...], preferred_element_type=jnp.float32) + b_ref[...]
  t_ref[...] = t

  @pl.when(i == 0)
  def _():
    st_ref[...] = jnp.zeros_like(st_ref)

  st_ref[0:1, :] += jnp.sum(t, axis=0, keepdims=True)
  st_ref[1:2, :] += jnp.sum(t * t, axis=0, keepdims=True)


@jax.jit
def _p1(x, agg, W1, b1):
  grid = (N // _RBLK,)
  return pl.pallas_call(
      _p1_kernel,
      grid=grid,
      in_specs=[
          pl.BlockSpec((_RBLK, D), lambda i: (i, 0)),
          pl.BlockSpec((1, _RBLK, D), lambda i: (0, i, 0)),
          pl.BlockSpec((1, _RBLK, D), lambda i: (1, i, 0)),
          pl.BlockSpec((D, D), lambda i: (0, 0)),
          pl.BlockSpec((1, D), lambda i: (0, 0)),
      ],
      out_specs=[
          pl.BlockSpec((_RBLK, D), lambda i: (i, 0)),
          pl.BlockSpec((2, D), lambda i: (0, 0)),
      ],
      out_shape=[
          jax.ShapeDtypeStruct((N, D), jnp.float32),
          jax.ShapeDtypeStruct((2, D), jnp.float32),
      ],
  )(x, agg, agg, W1, b1.reshape(1, D))


def _bn_relu(t_ref, st_ref, g_ref, bt_ref):
  mu = st_ref[0:1, :] * (1.0 / N)
  var = st_ref[1:2, :] * (1.0 / N) - mu * mu
  inv = lax.rsqrt(var + BN_EPS) * g_ref[...]
  return jnp.maximum((t_ref[...] - mu) * inv + bt_ref[...], 0.0)


def _p2a_kernel(t_ref, st_ref, g_ref, bt_ref, w2_ref, b2_ref, o_ref):
  r = _bn_relu(t_ref, st_ref, g_ref, bt_ref)
  h = jnp.dot(r, w2_ref[...], preferred_element_type=jnp.float32) + b2_ref[...]
  o_ref[...] = jnp.maximum(h, 0.0)


@jax.jit
def _p2a(t, st, g, bt, W2, b2):
  grid = (N // _RBLK,)
  wspec = pl.BlockSpec((D, D), lambda i: (0, 0))
  bspec = pl.BlockSpec((1, D), lambda i: (0, 0))
  return pl.pallas_call(
      _p2a_kernel,
      grid=grid,
      in_specs=[
          pl.BlockSpec((_RBLK, D), lambda i: (i, 0)),
          pl.BlockSpec((2, D), lambda i: (0, 0)),
          bspec, bspec, wspec, bspec,
      ],
      out_specs=pl.BlockSpec((_RBLK, D), lambda i: (i, 0)),
      out_shape=jax.ShapeDtypeStruct((N, D), jnp.float32),
  )(t, st, g.reshape(1, D), bt.reshape(1, D), W2, b2.reshape(1, D))


def _p2b_kernel(t_ref, st_ref, g_ref, bt_ref, w2_ref, b2_ref,
                wc1_ref, bc1_ref, wc2_ref, bc2_ref, o_ref):
  r = _bn_relu(t_ref, st_ref, g_ref, bt_ref)
  h = jnp.dot(r, w2_ref[...], preferred_element_type=jnp.float32) + b2_ref[...]
  h = jnp.maximum(h, 0.0)
  h = jnp.dot(h, wc1_ref[...], preferred_element_type=jnp.float32) + bc1_ref[...]
  h = jnp.maximum(h, 0.0)
  o_ref[...] = jnp.dot(h, wc2_ref[...],
                       preferred_element_type=jnp.float32) + bc2_ref[...]


@jax.jit
def _p2b(t, st, g, bt, W2, b2, Wc1, bc1, Wc2, bc2):
  grid = (N // _RBLK,)
  wspec = pl.BlockSpec((D, D), lambda i: (0, 0))
  bspec = pl.BlockSpec((1, D), lambda i: (0, 0))
  return pl.pallas_call(
      _p2b_kernel,
      grid=grid,
      in_specs=[
          pl.BlockSpec((_RBLK, D), lambda i: (i, 0)),
          pl.BlockSpec((2, D), lambda i: (0, 0)),
          bspec, bspec, wspec, bspec, wspec, bspec, wspec, bspec,
      ],
      out_specs=pl.BlockSpec((_RBLK, D), lambda i: (i, 0)),
      out_shape=jax.ShapeDtypeStruct((N, D), jnp.float32),
  )(t, st, g.reshape(1, D), bt.reshape(1, D), W2, b2.reshape(1, D),
    Wc1, bc1.reshape(1, D), Wc2, bc2.reshape(1, D))


def _p1_sum(x, agg_parts, W1, b1):
  return _p1(x, agg_parts, W1, b1)


def kernel(x, edge_index, edge_attr,
           We0, be0, W10, b10, g0, bt0, W20, b20,
           We1, be1, W11, b11, g1, bt1, W21, b21,
           Wc1, bc1, Wc2, bc2):
  src = edge_index[0]
  dst = edge_index[1]
  e0, e1 = _emb(edge_attr, We0, be0, We1, be1)

  agg0 = _sc_agg(x, src, dst, e0)
  t0, st0 = _p1_sum(x, agg0, W10, b10)
  x1 = _p2a(t0, st0, g0, bt0, W20, b20)

  agg1 = _sc_agg(x1, src, dst, e1)
  t1, st1 = _p1_sum(x1, agg1, W11, b11)
  out = _p2b(t1, st1, g1, bt1, W21, b21, Wc1, bc1, Wc2, bc2)
  return out


# speedup vs baseline: 3.6652x; 1.3477x over previous
import functools

import jax
import jax.numpy as jnp
from jax import lax
from jax.experimental import pallas as pl
from jax.experimental.pallas import tpu as pltpu
from jax.experimental.pallas import tpu_sc as plsc

N = 10000
E = 320000
D = 128
DE = 16
BN_EPS = 1e-5

NC = 2
NS = 16
EPT = E // (NC * NS)
EB = 80
NBLK = EPT // EB
NP = 10240
RPT = NP // NS


def _sc_agg_body(x_hbm, src_hbm, dst_hbm, e_hbm, zeros_hbm, out_hbm,
                 agg_sh, src0, src1, dst0, dst1, xb0, xb1, eb0, eb1,
                 gs0, gs1, es0, es1):
  c = lax.axis_index("c")
  s = lax.axis_index("s")
  srcs, dsts, xbs, ebs = (src0, src1), (dst0, dst1), (xb0, xb1), (eb0, eb1)
  gsems, esems = (gs0, gs1), (es0, es1)

  pltpu.sync_copy(zeros_hbm.at[:], agg_sh.at[pl.ds(s * RPT, RPT), :])
  plsc.subcore_barrier()

  base0 = (c * NS + s) * EPT

  def issue(i, b):
    base = base0 + i * EB
    pltpu.sync_copy(src_hbm.at[pl.ds(base, EB)], srcs[b])
    pltpu.sync_copy(dst_hbm.at[pl.ds(base, EB)], dsts[b])
    pltpu.async_copy(x_hbm.at[srcs[b]], xbs[b], gsems[b])
    pltpu.async_copy(e_hbm.at[pl.ds(base, EB), :], ebs[b], esems[b])

  def consume(b):
    pltpu.make_async_copy(x_hbm.at[srcs[b]], xbs[b], gsems[b]).wait()
    pltpu.make_async_copy(e_hbm.at[pl.ds(0, EB), :], ebs[b], esems[b]).wait()
    xbuf, ebuf = xbs[b], ebs[b]

    def row(r, carry2):
      for k in range(D // 16):
        sl = pl.ds(k * 16, 16)
        v = xbuf[r, sl] + ebuf[r, sl]
        xbuf[r, sl] = jnp.maximum(v, 0.0)
      return carry2

    lax.fori_loop(0, EB, row, 0, unroll=False)
    pltpu.sync_copy(xbuf, agg_sh.at[dsts[b]], add=True)

  issue(0, 0)

  def body(ii, carry):
    for b in range(2):
      i = ii * 2 + b

      @pl.when(i + 1 < NBLK)
      def _():
        issue(i + 1, 1 - b)

      consume(b)
    return carry

  lax.fori_loop(0, NBLK // 2, body, 0, unroll=False)
  if NBLK % 2:
    consume(0)
  plsc.subcore_barrier()
  pltpu.sync_copy(agg_sh.at[pl.ds(s * RPT, RPT), :],
                  out_hbm.at[c, pl.ds(s * RPT, RPT), :])


@jax.jit
def _sc_agg(x, src, dst, e):
  zeros = jnp.zeros((RPT, D), jnp.float32)
  fn = pl.kernel(
      _sc_agg_body,
      out_type=jax.ShapeDtypeStruct((NC, NP, D), jnp.float32),
      mesh=plsc.VectorSubcoreMesh(core_axis_name="c", subcore_axis_name="s"),
      scratch_types=[
          pltpu.VMEM_SHARED((NP, D), jnp.float32),
          pltpu.VMEM((EB,), jnp.int32),
          pltpu.VMEM((EB,), jnp.int32),
          pltpu.VMEM((EB,), jnp.int32),
          pltpu.VMEM((EB,), jnp.int32),
          pltpu.VMEM((EB, D), jnp.float32),
          pltpu.VMEM((EB, D), jnp.float32),
          pltpu.VMEM((EB, D), jnp.float32),
          pltpu.VMEM((EB, D), jnp.float32),
          pltpu.SemaphoreType.DMA,
          pltpu.SemaphoreType.DMA,
          pltpu.SemaphoreType.DMA,
          pltpu.SemaphoreType.DMA,
      ],
  )
  return fn(x, src, dst, e, zeros)


_EBLK = 2000
_RBLK = 1000


def _emb_kernel(ea_ref, we0_ref, be0_ref, we1_ref, be1_ref, e0_ref, e1_ref):
  ea = ea_ref[...]
  e0_ref[...] = jnp.dot(ea, we0_ref[...],
                        preferred_element_type=jnp.float32) + be0_ref[...]
  e1_ref[...] = jnp.dot(ea, we1_ref[...],
                        preferred_element_type=jnp.float32) + be1_ref[...]


@jax.jit
def _emb(edge_attr, We0, be0, We1, be1):
  grid = (E // _EBLK,)
  return pl.pallas_call(
      _emb_kernel,
      grid=grid,
      in_specs=[
          pl.BlockSpec((_EBLK, DE), lambda i: (i, 0)),
          pl.BlockSpec((DE, D), lambda i: (0, 0)),
          pl.BlockSpec((1, D), lambda i: (0, 0)),
          pl.BlockSpec((DE, D), lambda i: (0, 0)),
          pl.BlockSpec((1, D), lambda i: (0, 0)),
      ],
      out_specs=[
          pl.BlockSpec((_EBLK, D), lambda i: (i, 0)),
          pl.BlockSpec((_EBLK, D), lambda i: (i, 0)),
      ],
      out_shape=[
          jax.ShapeDtypeStruct((E, D), jnp.float32),
          jax.ShapeDtypeStruct((E, D), jnp.float32),
      ],
  )(edge_attr, We0, be0.reshape(1, D), We1, be1.reshape(1, D))


def _p1_kernel(x_ref, a0_ref, a1_ref, w_ref, b_ref, t_ref, st_ref):
  i = pl.program_id(0)
  h = x_ref[...] + a0_ref[0] + a1_ref[0]
  t = jnp.dot(h, w_ref[...], preferred_element_type=jnp.float32) + b_ref[...]
  t_ref[...] = t

  @pl.when(i == 0)
  def _():
    st_ref[...] = jnp.zeros_like(st_ref)

  st_ref[0:1, :] += jnp.sum(t, axis=0, keepdims=True)
  st_ref[1:2, :] += jnp.sum(t * t, axis=0, keepdims=True)


@jax.jit
def _p1(x, agg, W1, b1):
  grid = (N // _RBLK,)
  return pl.pallas_call(
      _p1_kernel,
      grid=grid,
      in_specs=[
          pl.BlockSpec((_RBLK, D), lambda i: (i, 0)),
          pl.BlockSpec((1, _RBLK, D), lambda i: (0, i, 0)),
          pl.BlockSpec((1, _RBLK, D), lambda i: (1, i, 0)),
          pl.BlockSpec((D, D), lambda i: (0, 0)),
          pl.BlockSpec((1, D), lambda i: (0, 0)),
      ],
      out_specs=[
          pl.BlockSpec((_RBLK, D), lambda i: (i, 0)),
          pl.BlockSpec((2, D), lambda i: (0, 0)),
      ],
      out_shape=[
          jax.ShapeDtypeStruct((N, D), jnp.float32),
          jax.ShapeDtypeStruct((2, D), jnp.float32),
      ],
  )(x, agg, agg, W1, b1.reshape(1, D))


def _bn_relu(t_ref, st_ref, g_ref, bt_ref):
  mu = st_ref[0:1, :] * (1.0 / N)
  var = st_ref[1:2, :] * (1.0 / N) - mu * mu
  inv = lax.rsqrt(var + BN_EPS) * g_ref[...]
  return jnp.maximum((t_ref[...] - mu) * inv + bt_ref[...], 0.0)


def _p2a_kernel(t_ref, st_ref, g_ref, bt_ref, w2_ref, b2_ref, o_ref):
  r = _bn_relu(t_ref, st_ref, g_ref, bt_ref)
  h = jnp.dot(r, w2_ref[...], preferred_element_type=jnp.float32) + b2_ref[...]
  o_ref[...] = jnp.maximum(h, 0.0)


@jax.jit
def _p2a(t, st, g, bt, W2, b2):
  grid = (N // _RBLK,)
  wspec = pl.BlockSpec((D, D), lambda i: (0, 0))
  bspec = pl.BlockSpec((1, D), lambda i: (0, 0))
  return pl.pallas_call(
      _p2a_kernel,
      grid=grid,
      in_specs=[
          pl.BlockSpec((_RBLK, D), lambda i: (i, 0)),
          pl.BlockSpec((2, D), lambda i: (0, 0)),
          bspec, bspec, wspec, bspec,
      ],
      out_specs=pl.BlockSpec((_RBLK, D), lambda i: (i, 0)),
      out_shape=jax.ShapeDtypeStruct((N, D), jnp.float32),
  )(t, st, g.reshape(1, D), bt.reshape(1, D), W2, b2.reshape(1, D))


def _p2b_kernel(t_ref, st_ref, g_ref, bt_ref, w2_ref, b2_ref,
                wc1_ref, bc1_ref, wc2_ref, bc2_ref, o_ref):
  r = _bn_relu(t_ref, st_ref, g_ref, bt_ref)
  h = jnp.dot(r, w2_ref[...], preferred_element_type=jnp.float32) + b2_ref[...]
  h = jnp.maximum(h, 0.0)
  h = jnp.dot(h, wc1_ref[...], preferred_element_type=jnp.float32) + bc1_ref[...]
  h = jnp.maximum(h, 0.0)
  o_ref[...] = jnp.dot(h, wc2_ref[...],
                       preferred_element_type=jnp.float32) + bc2_ref[...]


@jax.jit
def _p2b(t, st, g, bt, W2, b2, Wc1, bc1, Wc2, bc2):
  grid = (N // _RBLK,)
  wspec = pl.BlockSpec((D, D), lambda i: (0, 0))
  bspec = pl.BlockSpec((1, D), lambda i: (0, 0))
  return pl.pallas_call(
      _p2b_kernel,
      grid=grid,
      in_specs=[
          pl.BlockSpec((_RBLK, D), lambda i: (i, 0)),
          pl.BlockSpec((2, D), lambda i: (0, 0)),
          bspec, bspec, wspec, bspec, wspec, bspec, wspec, bspec,
      ],
      out_specs=pl.BlockSpec((_RBLK, D), lambda i: (i, 0)),
      out_shape=jax.ShapeDtypeStruct((N, D), jnp.float32),
  )(t, st, g.reshape(1, D), bt.reshape(1, D), W2, b2.reshape(1, D),
    Wc1, bc1.reshape(1, D), Wc2, bc2.reshape(1, D))


def _p1_sum(x, agg_parts, W1, b1):
  return _p1(x, agg_parts, W1, b1)


def kernel(x, edge_index, edge_attr,
           We0, be0, W10, b10, g0, bt0, W20, b20,
           We1, be1, W11, b11, g1, bt1, W21, b21,
           Wc1, bc1, Wc2, bc2):
  src = edge_index[0]
  dst = edge_index[1]
  e0, e1 = _emb(edge_attr, We0, be0, We1, be1)

  agg0 = _sc_agg(x, src, dst, e0)
  t0, st0 = _p1_sum(x, agg0, W10, b10)
  x1 = _p2a(t0, st0, g0, bt0, W20, b20)

  agg1 = _sc_agg(x1, src, dst, e1)
  t1, st1 = _p1_sum(x1, agg1, W11, b11)
  out = _p2b(t1, st1, g1, bt1, W21, b21, Wc1, bc1, Wc2, bc2)
  return out


# speedup vs baseline: 4.4222x; 1.2065x over previous
import functools

import jax
import jax.numpy as jnp
from jax import lax
from jax.experimental import pallas as pl
from jax.experimental.pallas import tpu as pltpu
from jax.experimental.pallas import tpu_sc as plsc

N = 10000
E = 320000
D = 128
DE = 16
BN_EPS = 1e-5

NC = 2
NS = 16
EPT = E // (NC * NS)
EB = 40
NBLK = EPT // EB
RING = 4
IRING = 8
NP = 10240
RPT = NP // NS


def _sc_agg_body(x_hbm, src_hbm, dst_hbm, e_hbm, zeros_hbm, out_hbm,
                 agg_sh, *rest):
  srcv = rest[0:8]
  dstv = rest[8:16]
  xbs = rest[16:20]
  ebs = rest[20:24]
  gsems = rest[24:28]
  esems = rest[28:32]
  ssems = rest[32:36]
  ipair = rest[36:40]
  c = lax.axis_index("c")
  s = lax.axis_index("s")

  tid = c * NS + s
  base0 = tid * EPT

  def idx_start(t, u):
    base = base0 + t * EB
    pltpu.async_copy(src_hbm.at[pl.ds(base, EB)], srcv[u], ipair[u % 4])
    pltpu.async_copy(dst_hbm.at[pl.ds(base, EB)], dstv[u], ipair[u % 4])

  def idx_wait(u):
    pltpu.make_async_copy(src_hbm.at[pl.ds(0, EB)], srcv[u],
                          ipair[u % 4]).wait()
    pltpu.make_async_copy(dst_hbm.at[pl.ds(0, EB)], dstv[u],
                          ipair[u % 4]).wait()

  def gather_start(t, u):
    b = u % RING
    pltpu.async_copy(x_hbm.at[srcv[u]], xbs[b], gsems[b])
    pltpu.async_copy(e_hbm.at[pl.ds(base0 + t * EB, EB), :], ebs[b], esems[b])

  def scatter_wait(m):
    pltpu.make_async_copy(xbs[0], agg_sh.at[dstv[0]], ssems[m]).wait()

  def consume(u):
    b = u % RING
    pltpu.make_async_copy(x_hbm.at[srcv[0]], xbs[b], gsems[b]).wait()
    pltpu.make_async_copy(e_hbm.at[pl.ds(0, EB), :], ebs[b], esems[b]).wait()
    xbuf, ebuf = xbs[b], ebs[b]

    def row(r, carry2):
      for k in range(D // 16):
        sl = pl.ds(k * 16, 16)
        v = xbuf[r, sl] + ebuf[r, sl]
        xbuf[r, sl] = jnp.maximum(v, 0.0)
      return carry2

    lax.fori_loop(0, EB, row, 0, unroll=False)
    pltpu.async_copy(xbuf, agg_sh.at[dstv[u]], ssems[u % 4], add=True)

  idx_start(0, 0)
  idx_start(1, 1)
  pltpu.sync_copy(zeros_hbm.at[:], agg_sh.at[pl.ds(s * RPT, RPT), :])
  plsc.subcore_barrier()
  idx_wait(0)
  gather_start(0, 0)

  def octet(tt, carry):
    for u in range(IRING):
      t = tt * IRING + u
      if u < 3:
        @pl.when(tt >= 1)
        def _():
          scatter_wait((u + 1) % 4)
      else:
        scatter_wait((u + 1) % 4)
      idx_start(t + 2, (u + 2) % IRING)
      idx_wait((u + 1) % IRING)
      gather_start(t + 1, (u + 1) % IRING)
      consume(u)
    return carry

  lax.fori_loop(0, NBLK // IRING, octet, 0, unroll=False)
  tb = (NBLK // IRING) * IRING
  scatter_wait(1)
  idx_wait(1)
  gather_start(tb + 1, 1)
  consume(0)
  scatter_wait(2)
  consume(1)
  scatter_wait(3)
  scatter_wait(0)
  scatter_wait(1)
  plsc.subcore_barrier()
  pltpu.sync_copy(agg_sh.at[pl.ds(s * RPT, RPT), :],
                  out_hbm.at[c, pl.ds(s * RPT, RPT), :])


@jax.jit
def _sc_agg(x, src, dst, e):
  zeros = jnp.zeros((RPT, D), jnp.float32)
  fn = pl.kernel(
      _sc_agg_body,
      out_type=jax.ShapeDtypeStruct((NC, NP, D), jnp.float32),
      mesh=plsc.VectorSubcoreMesh(core_axis_name="c", subcore_axis_name="s"),
      scratch_types=[
          pltpu.VMEM_SHARED((NP, D), jnp.float32),
      ] + [pltpu.VMEM((EB,), jnp.int32)] * (2 * IRING)
        + [pltpu.VMEM((EB, D), jnp.float32)] * (2 * RING)
        + [pltpu.SemaphoreType.DMA] * 16,
  )
  return fn(x, src, dst, e, zeros)


_EBLK = 2000
_RBLK = 1000


def _emb_kernel(ea_ref, we0_ref, be0_ref, we1_ref, be1_ref, e0_ref, e1_ref):
  ea = ea_ref[...]
  e0_ref[...] = jnp.dot(ea, we0_ref[...],
                        preferred_element_type=jnp.float32) + be0_ref[...]
  e1_ref[...] = jnp.dot(ea, we1_ref[...],
                        preferred_element_type=jnp.float32) + be1_ref[...]


@jax.jit
def _emb(edge_attr, We0, be0, We1, be1):
  grid = (E // _EBLK,)
  return pl.pallas_call(
      _emb_kernel,
      grid=grid,
      in_specs=[
          pl.BlockSpec((_EBLK, DE), lambda i: (i, 0)),
          pl.BlockSpec((DE, D), lambda i: (0, 0)),
          pl.BlockSpec((1, D), lambda i: (0, 0)),
          pl.BlockSpec((DE, D), lambda i: (0, 0)),
          pl.BlockSpec((1, D), lambda i: (0, 0)),
      ],
      out_specs=[
          pl.BlockSpec((_EBLK, D), lambda i: (i, 0)),
          pl.BlockSpec((_EBLK, D), lambda i: (i, 0)),
      ],
      out_shape=[
          jax.ShapeDtypeStruct((E, D), jnp.float32),
          jax.ShapeDtypeStruct((E, D), jnp.float32),
      ],
  )(edge_attr, We0, be0.reshape(1, D), We1, be1.reshape(1, D))


def _p1_kernel(x_ref, a0_ref, a1_ref, w_ref, b_ref, t_ref, st_ref):
  i = pl.program_id(0)
  h = x_ref[...] + a0_ref[0] + a1_ref[0]
  t = jnp.dot(h, w_ref[...], preferred_element_type=jnp.float32) + b_ref[...]
  t_ref[...] = t

  @pl.when(i == 0)
  def _():
    st_ref[...] = jnp.zeros_like(st_ref)

  st_ref[0:1, :] += jnp.sum(t, axis=0, keepdims=True)
  st_ref[1:2, :] += jnp.sum(t * t, axis=0, keepdims=True)


@jax.jit
def _p1(x, agg, W1, b1):
  grid = (N // _RBLK,)
  return pl.pallas_call(
      _p1_kernel,
      grid=grid,
      in_specs=[
          pl.BlockSpec((_RBLK, D), lambda i: (i, 0)),
          pl.BlockSpec((1, _RBLK, D), lambda i: (0, i, 0)),
          pl.BlockSpec((1, _RBLK, D), lambda i: (1, i, 0)),
          pl.BlockSpec((D, D), lambda i: (0, 0)),
          pl.BlockSpec((1, D), lambda i: (0, 0)),
      ],
      out_specs=[
          pl.BlockSpec((_RBLK, D), lambda i: (i, 0)),
          pl.BlockSpec((2, D), lambda i: (0, 0)),
      ],
      out_shape=[
          jax.ShapeDtypeStruct((N, D), jnp.float32),
          jax.ShapeDtypeStruct((2, D), jnp.float32),
      ],
  )(x, agg, agg, W1, b1.reshape(1, D))


def _bn_relu(t_ref, st_ref, g_ref, bt_ref):
  mu = st_ref[0:1, :] * (1.0 / N)
  var = st_ref[1:2, :] * (1.0 / N) - mu * mu
  inv = lax.rsqrt(var + BN_EPS) * g_ref[...]
  return jnp.maximum((t_ref[...] - mu) * inv + bt_ref[...], 0.0)


def _p2a_kernel(t_ref, st_ref, g_ref, bt_ref, w2_ref, b2_ref, o_ref):
  r = _bn_relu(t_ref, st_ref, g_ref, bt_ref)
  h = jnp.dot(r, w2_ref[...], preferred_element_type=jnp.float32) + b2_ref[...]
  o_ref[...] = jnp.maximum(h, 0.0)


@jax.jit
def _p2a(t, st, g, bt, W2, b2):
  grid = (N // _RBLK,)
  wspec = pl.BlockSpec((D, D), lambda i: (0, 0))
  bspec = pl.BlockSpec((1, D), lambda i: (0, 0))
  return pl.pallas_call(
      _p2a_kernel,
      grid=grid,
      in_specs=[
          pl.BlockSpec((_RBLK, D), lambda i: (i, 0)),
          pl.BlockSpec((2, D), lambda i: (0, 0)),
          bspec, bspec, wspec, bspec,
      ],
      out_specs=pl.BlockSpec((_RBLK, D), lambda i: (i, 0)),
      out_shape=jax.ShapeDtypeStruct((N, D), jnp.float32),
  )(t, st, g.reshape(1, D), bt.reshape(1, D), W2, b2.reshape(1, D))


def _p2b_kernel(t_ref, st_ref, g_ref, bt_ref, w2_ref, b2_ref,
                wc1_ref, bc1_ref, wc2_ref, bc2_ref, o_ref):
  r = _bn_relu(t_ref, st_ref, g_ref, bt_ref)
  h = jnp.dot(r, w2_ref[...], preferred_element_type=jnp.float32) + b2_ref[...]
  h = jnp.maximum(h, 0.0)
  h = jnp.dot(h, wc1_ref[...], preferred_element_type=jnp.float32) + bc1_ref[...]
  h = jnp.maximum(h, 0.0)
  o_ref[...] = jnp.dot(h, wc2_ref[...],
                       preferred_element_type=jnp.float32) + bc2_ref[...]


@jax.jit
def _p2b(t, st, g, bt, W2, b2, Wc1, bc1, Wc2, bc2):
  grid = (N // _RBLK,)
  wspec = pl.BlockSpec((D, D), lambda i: (0, 0))
  bspec = pl.BlockSpec((1, D), lambda i: (0, 0))
  return pl.pallas_call(
      _p2b_kernel,
      grid=grid,
      in_specs=[
          pl.BlockSpec((_RBLK, D), lambda i: (i, 0)),
          pl.BlockSpec((2, D), lambda i: (0, 0)),
          bspec, bspec, wspec, bspec, wspec, bspec, wspec, bspec,
      ],
      out_specs=pl.BlockSpec((_RBLK, D), lambda i: (i, 0)),
      out_shape=jax.ShapeDtypeStruct((N, D), jnp.float32),
  )(t, st, g.reshape(1, D), bt.reshape(1, D), W2, b2.reshape(1, D),
    Wc1, bc1.reshape(1, D), Wc2, bc2.reshape(1, D))


def kernel(x, edge_index, edge_attr,
           We0, be0, W10, b10, g0, bt0, W20, b20,
           We1, be1, W11, b11, g1, bt1, W21, b21,
           Wc1, bc1, Wc2, bc2):
  src = edge_index[0]
  dst = edge_index[1]
  e0, e1 = _emb(edge_attr, We0, be0, We1, be1)

  agg0 = _sc_agg(x, src, dst, e0)
  t0, st0 = _p1(x, agg0, W10, b10)
  x1 = _p2a(t0, st0, g0, bt0, W20, b20)

  agg1 = _sc_agg(x1, src, dst, e1)
  t1, st1 = _p1(x1, agg1, W11, b11)
  out = _p2b(t1, st1, g1, bt1, W21, b21, Wc1, bc1, Wc2, bc2)
  return out
